# baseline (speedup 1.0000x reference)
import functools

import jax
import jax.numpy as jnp
from jax.experimental import pallas as pl
from jax.experimental.pallas import tpu as pltpu
from jax.experimental.pallas import tpu_sc as plsc


_V_BLK = 2048
_NBUF = 4
_NUM_WORKERS = 32


def _sc_gather(table, idx):
    n = idx.shape[0]
    e = table.shape[1]
    per_w = n // _NUM_WORKERS

    mesh = plsc.VectorSubcoreMesh(core_axis_name="c", subcore_axis_name="s")

    @functools.partial(
        pl.kernel,
        mesh=mesh,
        out_type=jax.ShapeDtypeStruct((n, e), table.dtype),
        scratch_types=[
            pltpu.VMEM((per_w,), jnp.int32),
            pltpu.VMEM((per_w, e), table.dtype),
            pltpu.SemaphoreType.DMA,
        ],
        compiler_params=pltpu.CompilerParams(use_tc_tiling_on_sc=False),
    )
    def gather_kernel(tbl_hbm, i_hbm, o_hbm, idx_v, rows_v, sem):
        wid = jax.lax.axis_index("s") * 2 + jax.lax.axis_index("c")
        base = wid * per_w
        pltpu.sync_copy(i_hbm.at[pl.ds(base, per_w)], idx_v)
        pltpu.async_copy(tbl_hbm.at[idx_v], rows_v, sem).wait()
        pltpu.sync_copy(rows_v, o_hbm.at[pl.ds(base, per_w)])

    return gather_kernel(table, idx)


def _make_mm_body(nsteps, tail, batch):
    def _mm_body(emb_ref, wt_ref, b_ref, o_hbm, obuf, osem):
        i = pl.program_id(0)
        s = jax.lax.rem(i, _NBUF)

        @pl.when(i >= _NBUF)
        def _():
            pltpu.make_async_copy(
                obuf.at[s],
                o_hbm.at[pl.ds((i - _NBUF) * _V_BLK, _V_BLK), :],
                osem.at[s],
            ).wait()

        obuf[s] = jax.lax.dot_general(
            wt_ref[...], emb_ref[...],
            dimension_numbers=(((0,), (1,)), ((), ())),
            preferred_element_type=jnp.float32,
        ) + b_ref[...]

        @pl.when(i < nsteps - 1)
        def _():
            pltpu.make_async_copy(
                obuf.at[s],
                o_hbm.at[pl.ds(i * _V_BLK, _V_BLK), :],
                osem.at[s],
            ).start()

        @pl.when(i == nsteps - 1)
        def _():
            last = nsteps - 1
            pltpu.make_async_copy(
                obuf.at[s, pl.ds(0, tail), :],
                o_hbm.at[pl.ds(last * _V_BLK, tail), :],
                osem.at[s],
            ).start()
            for j in range(max(0, nsteps - _NBUF), nsteps):
                sz = _V_BLK if j < nsteps - 1 else tail
                pltpu.make_async_copy(
                    obuf.at[j % _NBUF, pl.ds(0, sz), :],
                    o_hbm.at[pl.ds(j * _V_BLK, sz), :],
                    osem.at[j % _NBUF],
                ).wait()

    return _mm_body


def _tc_project_t(emb, Wt, bc):
    e, vocab = Wt.shape
    batch = emb.shape[0]
    nsteps = pl.cdiv(vocab, _V_BLK)
    tail = vocab - (nsteps - 1) * _V_BLK
    return pl.pallas_call(
        _make_mm_body(nsteps, tail, batch),
        grid=(nsteps,),
        in_specs=[
            pl.BlockSpec((batch, e), lambda i: (0, 0)),
            pl.BlockSpec((e, _V_BLK), lambda i: (0, i)),
            pl.BlockSpec((_V_BLK, 1), lambda i: (i, 0)),
        ],
        out_specs=pl.BlockSpec(memory_space=pltpu.MemorySpace.HBM),
        out_shape=jax.ShapeDtypeStruct((vocab, batch), jnp.float32),
        scratch_shapes=[
            pltpu.VMEM((_NBUF, _V_BLK, batch), jnp.float32),
            pltpu.SemaphoreType.DMA((_NBUF,)),
        ],
        compiler_params=pltpu.CompilerParams(
            dimension_semantics=("arbitrary",)),
    )(emb, Wt, bc)


def kernel(data, table, W, b):
    data = data.astype(jnp.int32)
    vocab, e = table.shape
    emb = _sc_gather(table, data)
    out_t = _tc_project_t(emb, jnp.transpose(W), b.reshape(vocab, 1))
    return jnp.transpose(out_t)


# speedup vs baseline: 1.0101x; 1.0101x over previous
import functools

import jax
import jax.numpy as jnp
from jax.experimental import pallas as pl
from jax.experimental.pallas import tpu as pltpu
from jax.experimental.pallas import tpu_sc as plsc


_V_BLK = 4096
_NUM_WORKERS = 32


def _sc_gather(table, idx):
    n = idx.shape[0]
    e = table.shape[1]
    per_w = n // _NUM_WORKERS

    mesh = plsc.VectorSubcoreMesh(core_axis_name="c", subcore_axis_name="s")

    @functools.partial(
        pl.kernel,
        mesh=mesh,
        out_type=jax.ShapeDtypeStruct((n, e), table.dtype),
        scratch_types=[
            pltpu.VMEM((per_w,), jnp.int32),
            pltpu.VMEM((per_w, e), table.dtype),
            pltpu.SemaphoreType.DMA,
        ],
        compiler_params=pltpu.CompilerParams(use_tc_tiling_on_sc=False),
    )
    def gather_kernel(tbl_hbm, i_hbm, o_hbm, idx_v, rows_v, sem):
        wid = jax.lax.axis_index("s") * 2 + jax.lax.axis_index("c")
        base = wid * per_w
        pltpu.sync_copy(i_hbm.at[pl.ds(base, per_w)], idx_v)
        pltpu.async_copy(tbl_hbm.at[idx_v], rows_v, sem).wait()
        pltpu.sync_copy(rows_v, o_hbm.at[pl.ds(base, per_w)])

    return gather_kernel(table, idx)


def _mm_body(emb_ref, wt_ref, b_ref, o_ref):
    o_ref[...] = jax.lax.dot_general(
        wt_ref[...], emb_ref[...],
        dimension_numbers=(((0,), (1,)), ((), ())),
        preferred_element_type=jnp.float32,
    ) + b_ref[...]


def _tc_project_t(emb, Wt, bc):
    e, vocab = Wt.shape
    batch = emb.shape[0]
    num_blocks = pl.cdiv(vocab, _V_BLK)
    return pl.pallas_call(
        _mm_body,
        grid=(num_blocks,),
        in_specs=[
            pl.BlockSpec((batch, e), lambda i: (0, 0)),
            pl.BlockSpec((e, _V_BLK), lambda i: (0, i)),
            pl.BlockSpec((_V_BLK, 1), lambda i: (i, 0)),
        ],
        out_specs=pl.BlockSpec((_V_BLK, batch), lambda i: (i, 0)),
        out_shape=jax.ShapeDtypeStruct((vocab, batch), jnp.float32),
        compiler_params=pltpu.CompilerParams(
            dimension_semantics=("parallel",)),
    )(emb, Wt, bc)


def kernel(data, table, W, b):
    data = data.astype(jnp.int32)
    vocab, e = table.shape
    emb = _sc_gather(table, data)
    out_t = _tc_project_t(emb, jnp.transpose(W), b.reshape(vocab, 1))
    return jnp.transpose(out_t)
